# baseline (speedup 1.0000x reference)
import functools

import jax
import jax.numpy as jnp
from jax import lax
from jax.experimental import pallas as pl
from jax.experimental.pallas import tpu as pltpu
from jax.experimental.pallas import tpu_sc as plsc

NC = 2
NS = 16
LANES = 16

B = 16384
L = 20
K = 20
E = 64
W = L + K
EV = E // LANES

NW = NC * NS
PER_W = B // NW
C = 8
CHUNKS = PER_W // C
IDX_PER_CHUNK = C * W
STREAM_W = 192
DMA_W = IDX_PER_CHUNK - STREAM_W
SLABS = (128, 64)
NBUF = 4
OUT_PAD = 32

_GDN = lax.GatherDimensionNumbers(
    offset_dims=(), collapsed_slice_dims=(0,), start_index_map=(0,))


def _perm(v, idx):
    return lax.gather(v, idx[:, None], _GDN, slice_sizes=(1,),
                      mode=lax.GatherScatterMode.PROMISE_IN_BOUNDS)


def _body(doc_idx_hbm, word_idx_hbm, doc_table_hbm, word_table_hbm, out_hbm,
          idx_w_all, idx_d_all,
          rows_d0, rows_d1, rows_d2, rows_d3,
          rows_w0, rows_w1, rows_w2, rows_w3,
          out_v0, out_v1, out_v2, out_v3,
          sem0, sem1, sem2, sem3,
          dsem0, dsem1, dsem2, dsem3):
    wid = lax.axis_index("s") * NC + lax.axis_index("c")
    wbase = wid * PER_W

    rows_d = (rows_d0, rows_d1, rows_d2, rows_d3)
    rows_w = (rows_w0, rows_w1, rows_w2, rows_w3)
    out_v = (out_v0, out_v1, out_v2, out_v3)
    sem = (sem0, sem1, sem2, sem3)
    dsem = (dsem0, dsem1, dsem2, dsem3)

    pltpu.sync_copy(
        word_idx_hbm.at[pl.ds(pl.multiple_of(wbase * W, 8), PER_W * W)],
        idx_w_all)
    pltpu.sync_copy(doc_idx_hbm.at[pl.ds(pl.multiple_of(wbase, 8), PER_W)],
                    idx_d_all.at[pl.ds(0, PER_W)])

    def issue(g, b):
        off = 0
        for n in SLABS:
            pltpu.make_async_copy(
                word_table_hbm.at[idx_w_all.at[pl.ds(g * IDX_PER_CHUNK + off,
                                                     n)]],
                rows_w[b].at[pl.ds(off, n)],
                sem[b]).start()
            off += n
        def dma_row(r, carry):
            v = idx_w_all[pl.ds(g * IDX_PER_CHUNK + STREAM_W + r * LANES,
                                LANES)]
            for j in range(LANES):
                pltpu.make_async_copy(
                    word_table_hbm.at[pl.ds(v[j], 1)],
                    rows_w[b].at[pl.ds(STREAM_W + r * LANES + j, 1)],
                    dsem[b]).start()
            return carry

        lax.fori_loop(0, DMA_W // LANES, dma_row, jnp.int32(0))

        vd = idx_d_all[pl.ds(g * C, LANES)]
        for j in range(C):
            pltpu.make_async_copy(
                doc_table_hbm.at[pl.ds(vd[j], 1)],
                rows_d[b].at[pl.ds(j, 1)],
                dsem[b]).start()

    def drain(g, b):
        off = 0
        for n in SLABS:
            pltpu.make_async_copy(
                word_table_hbm.at[idx_w_all.at[pl.ds(g * IDX_PER_CHUNK + off,
                                                     n)]],
                rows_w[b].at[pl.ds(off, n)],
                sem[b]).wait()
            off += n

        def dma_row_wait(r, carry):
            pltpu.make_async_copy(
                word_table_hbm.at[pl.ds(jnp.int32(0), 1)],
                rows_w[b].at[pl.ds(STREAM_W, 1)],
                dsem[b]).wait()
            return carry

        lax.fori_loop(0, DMA_W + C, dma_row_wait, jnp.int32(0))

    inv_n = jnp.float32(1.0 / (L + 1))
    lane = lax.broadcasted_iota(jnp.int32, (LANES,), 0)
    rot_idx = {d: (lane + d) & (LANES - 1) for d in (1, 2, 4, 8)}

    def compute(g, b):
        rd, rw, ov = rows_d[b], rows_w[b], out_v[b]

        def ex_body(i, carry):
            rb = i * W
            m = [rd[i, pl.ds(LANES * e, LANES)] for e in range(EV)]
            for j in range(L):
                for e in range(EV):
                    m[e] = m[e] + rw[rb + j, pl.ds(LANES * e, LANES)]
            m = [v * inv_n for v in m]
            acc_a = jnp.zeros((LANES,), jnp.float32)
            acc_b = jnp.zeros((LANES,), jnp.float32)
            for k in range(K):
                r = rb + L + k
                p = m[0] * rw[r, pl.ds(0, LANES)]
                for e in range(1, EV):
                    p = p + m[e] * rw[r, pl.ds(LANES * e, LANES)]
                for d in (8, 4, 2, 1):
                    p = p + _perm(p, rot_idx[d])
                if k < LANES:
                    acc_a = jnp.where(lane == k, p, acc_a)
                else:
                    acc_b = jnp.where(lane == (k - LANES), p, acc_b)
            ov[i, pl.ds(0, LANES)] = acc_a
            ov[i, pl.ds(LANES, LANES)] = acc_b
            return carry

        lax.fori_loop(0, C, ex_body, jnp.int32(0))
        base = pl.multiple_of(wbase + g * C, 8)
        pltpu.sync_copy(ov, out_hbm.at[pl.ds(base, C)])

    for b in range(NBUF):
        issue(b, b)

    def it_body(it, carry):
        for b in range(NBUF):
            g = NBUF * it + b
            drain(g, b)
            compute(g, b)

            @pl.when(g + NBUF < CHUNKS)
            def _():
                issue(g + NBUF, b)
        return carry

    lax.fori_loop(0, CHUNKS // NBUF, it_body, jnp.int32(0))


@jax.jit
def _pv_dm(doc_idx, word_idx, doc_table, word_table):
    mesh = plsc.VectorSubcoreMesh(core_axis_name="c", subcore_axis_name="s")
    fn = pl.kernel(
        _body,
        mesh=mesh,
        compiler_params=pltpu.CompilerParams(use_tc_tiling_on_sc=False),
        out_type=jax.ShapeDtypeStruct((B, OUT_PAD), jnp.float32),
        scratch_types=[
            pltpu.VMEM((PER_W * W,), jnp.int32),
            pltpu.VMEM((PER_W + LANES,), jnp.int32),
            pltpu.VMEM((C, E), jnp.float32),
            pltpu.VMEM((C, E), jnp.float32),
            pltpu.VMEM((C, E), jnp.float32),
            pltpu.VMEM((C, E), jnp.float32),
            pltpu.VMEM((IDX_PER_CHUNK, E), jnp.float32),
            pltpu.VMEM((IDX_PER_CHUNK, E), jnp.float32),
            pltpu.VMEM((IDX_PER_CHUNK, E), jnp.float32),
            pltpu.VMEM((IDX_PER_CHUNK, E), jnp.float32),
            pltpu.VMEM((C, OUT_PAD), jnp.float32),
            pltpu.VMEM((C, OUT_PAD), jnp.float32),
            pltpu.VMEM((C, OUT_PAD), jnp.float32),
            pltpu.VMEM((C, OUT_PAD), jnp.float32),
            pltpu.SemaphoreType.DMA,
            pltpu.SemaphoreType.DMA,
            pltpu.SemaphoreType.DMA,
            pltpu.SemaphoreType.DMA,
            pltpu.SemaphoreType.DMA,
            pltpu.SemaphoreType.DMA,
            pltpu.SemaphoreType.DMA,
            pltpu.SemaphoreType.DMA,
        ],
    )
    return fn(doc_idx, word_idx, doc_table, word_table)


def kernel(doc_ids, context_ids, target_noise_ids, doc_table, word_table):
    doc_idx = doc_ids.reshape(B).astype(jnp.int32)
    word_idx = jnp.concatenate(
        [context_ids, target_noise_ids], axis=1).astype(jnp.int32)
    word_idx = word_idx.reshape(B * W)
    out = _pv_dm(doc_idx, word_idx, doc_table, word_table)
    return out[:, :K]


# speedup vs baseline: 1.0324x; 1.0324x over previous
import functools

import jax
import jax.numpy as jnp
from jax import lax
from jax.experimental import pallas as pl
from jax.experimental.pallas import tpu as pltpu
from jax.experimental.pallas import tpu_sc as plsc

NC = 2
NS = 16
LANES = 16

B = 16384
L = 20
K = 20
E = 64
W = L + K
EV = E // LANES

NW = NC * NS
PER_W = B // NW
C = 8
CHUNKS = PER_W // C
IDX_PER_CHUNK = C * W
STREAM_W = 256
DMA_W = IDX_PER_CHUNK - STREAM_W
SLABS = (128, 128)
NBUF = 4
OUT_PAD = 32

_GDN = lax.GatherDimensionNumbers(
    offset_dims=(), collapsed_slice_dims=(0,), start_index_map=(0,))


def _perm(v, idx):
    return lax.gather(v, idx[:, None], _GDN, slice_sizes=(1,),
                      mode=lax.GatherScatterMode.PROMISE_IN_BOUNDS)


def _body(doc_idx_hbm, word_idx_hbm, doc_table_hbm, word_table_hbm, out_hbm,
          idx_w_all, idx_d_all,
          rows_d0, rows_d1, rows_d2, rows_d3,
          rows_w0, rows_w1, rows_w2, rows_w3,
          out_v0, out_v1, out_v2, out_v3,
          sem0, sem1, sem2, sem3,
          dsem0, dsem1, dsem2, dsem3):
    wid = lax.axis_index("s") * NC + lax.axis_index("c")
    wbase = wid * PER_W

    rows_d = (rows_d0, rows_d1, rows_d2, rows_d3)
    rows_w = (rows_w0, rows_w1, rows_w2, rows_w3)
    out_v = (out_v0, out_v1, out_v2, out_v3)
    sem = (sem0, sem1, sem2, sem3)
    dsem = (dsem0, dsem1, dsem2, dsem3)

    pltpu.sync_copy(
        word_idx_hbm.at[pl.ds(pl.multiple_of(wbase * W, 8), PER_W * W)],
        idx_w_all)
    pltpu.sync_copy(doc_idx_hbm.at[pl.ds(pl.multiple_of(wbase, 8), PER_W)],
                    idx_d_all.at[pl.ds(0, PER_W)])

    def issue(g, b):
        off = 0
        for n in SLABS:
            pltpu.make_async_copy(
                word_table_hbm.at[idx_w_all.at[pl.ds(g * IDX_PER_CHUNK + off,
                                                     n)]],
                rows_w[b].at[pl.ds(off, n)],
                sem[b]).start()
            off += n
        def dma_row(r, carry):
            v = idx_w_all[pl.ds(g * IDX_PER_CHUNK + STREAM_W + r * LANES,
                                LANES)]
            for j in range(LANES):
                pltpu.make_async_copy(
                    word_table_hbm.at[pl.ds(v[j], 1)],
                    rows_w[b].at[pl.ds(STREAM_W + r * LANES + j, 1)],
                    dsem[b]).start()
            return carry

        lax.fori_loop(0, DMA_W // LANES, dma_row, jnp.int32(0))

        vd = idx_d_all[pl.ds(g * C, LANES)]
        for j in range(C):
            pltpu.make_async_copy(
                doc_table_hbm.at[pl.ds(vd[j], 1)],
                rows_d[b].at[pl.ds(j, 1)],
                dsem[b]).start()

    def drain(g, b):
        off = 0
        for n in SLABS:
            pltpu.make_async_copy(
                word_table_hbm.at[idx_w_all.at[pl.ds(g * IDX_PER_CHUNK + off,
                                                     n)]],
                rows_w[b].at[pl.ds(off, n)],
                sem[b]).wait()
            off += n

        def dma_row_wait(r, carry):
            pltpu.make_async_copy(
                word_table_hbm.at[pl.ds(jnp.int32(0), 1)],
                rows_w[b].at[pl.ds(STREAM_W, 1)],
                dsem[b]).wait()
            return carry

        lax.fori_loop(0, DMA_W + C, dma_row_wait, jnp.int32(0))

    inv_n = jnp.float32(1.0 / (L + 1))
    lane = lax.broadcasted_iota(jnp.int32, (LANES,), 0)
    rot_idx = {d: (lane + d) & (LANES - 1) for d in (1, 2, 4, 8)}

    def compute(g, b):
        rd, rw, ov = rows_d[b], rows_w[b], out_v[b]

        def ex_body(i, carry):
            rb = i * W
            m = [rd[i, pl.ds(LANES * e, LANES)] for e in range(EV)]
            for j in range(L):
                for e in range(EV):
                    m[e] = m[e] + rw[rb + j, pl.ds(LANES * e, LANES)]
            m = [v * inv_n for v in m]
            acc_a = jnp.zeros((LANES,), jnp.float32)
            acc_b = jnp.zeros((LANES,), jnp.float32)
            for k in range(K):
                r = rb + L + k
                p = m[0] * rw[r, pl.ds(0, LANES)]
                for e in range(1, EV):
                    p = p + m[e] * rw[r, pl.ds(LANES * e, LANES)]
                for d in (8, 4, 2, 1):
                    p = p + _perm(p, rot_idx[d])
                if k < LANES:
                    acc_a = jnp.where(lane == k, p, acc_a)
                else:
                    acc_b = jnp.where(lane == (k - LANES), p, acc_b)
            ov[i, pl.ds(0, LANES)] = acc_a
            ov[i, pl.ds(LANES, LANES)] = acc_b
            return carry

        lax.fori_loop(0, C, ex_body, jnp.int32(0))
        base = pl.multiple_of(wbase + g * C, 8)
        pltpu.sync_copy(ov, out_hbm.at[pl.ds(base, C)])

    for b in range(NBUF):
        issue(b, b)

    def it_body(it, carry):
        for b in range(NBUF):
            g = NBUF * it + b
            drain(g, b)
            compute(g, b)

            @pl.when(g + NBUF < CHUNKS)
            def _():
                issue(g + NBUF, b)
        return carry

    lax.fori_loop(0, CHUNKS // NBUF, it_body, jnp.int32(0))


@jax.jit
def _pv_dm(doc_idx, word_idx, doc_table, word_table):
    mesh = plsc.VectorSubcoreMesh(core_axis_name="c", subcore_axis_name="s")
    fn = pl.kernel(
        _body,
        mesh=mesh,
        compiler_params=pltpu.CompilerParams(use_tc_tiling_on_sc=False),
        out_type=jax.ShapeDtypeStruct((B, OUT_PAD), jnp.float32),
        scratch_types=[
            pltpu.VMEM((PER_W * W,), jnp.int32),
            pltpu.VMEM((PER_W + LANES,), jnp.int32),
            pltpu.VMEM((C, E), jnp.float32),
            pltpu.VMEM((C, E), jnp.float32),
            pltpu.VMEM((C, E), jnp.float32),
            pltpu.VMEM((C, E), jnp.float32),
            pltpu.VMEM((IDX_PER_CHUNK, E), jnp.float32),
            pltpu.VMEM((IDX_PER_CHUNK, E), jnp.float32),
            pltpu.VMEM((IDX_PER_CHUNK, E), jnp.float32),
            pltpu.VMEM((IDX_PER_CHUNK, E), jnp.float32),
            pltpu.VMEM((C, OUT_PAD), jnp.float32),
            pltpu.VMEM((C, OUT_PAD), jnp.float32),
            pltpu.VMEM((C, OUT_PAD), jnp.float32),
            pltpu.VMEM((C, OUT_PAD), jnp.float32),
            pltpu.SemaphoreType.DMA,
            pltpu.SemaphoreType.DMA,
            pltpu.SemaphoreType.DMA,
            pltpu.SemaphoreType.DMA,
            pltpu.SemaphoreType.DMA,
            pltpu.SemaphoreType.DMA,
            pltpu.SemaphoreType.DMA,
            pltpu.SemaphoreType.DMA,
        ],
    )
    return fn(doc_idx, word_idx, doc_table, word_table)


def kernel(doc_ids, context_ids, target_noise_ids, doc_table, word_table):
    doc_idx = doc_ids.reshape(B).astype(jnp.int32)
    word_idx = jnp.concatenate(
        [context_ids, target_noise_ids], axis=1).astype(jnp.int32)
    word_idx = word_idx.reshape(B * W)
    out = _pv_dm(doc_idx, word_idx, doc_table, word_table)
    return out[:, :K]


# speedup vs baseline: 1.0802x; 1.0463x over previous
import functools

import jax
import jax.numpy as jnp
from jax import lax
from jax.experimental import pallas as pl
from jax.experimental.pallas import tpu as pltpu
from jax.experimental.pallas import tpu_sc as plsc

NC = 2
NS = 16
LANES = 16

B = 16384
L = 20
K = 20
E = 64
W = L + K
EV = E // LANES

NW = NC * NS
PER_W = B // NW
C = 8
CHUNKS = PER_W // C
IDX_PER_CHUNK = C * W
STREAM_W = 320
DMA_W = IDX_PER_CHUNK - STREAM_W
SLABS = (128, 128, 64)
NBUF = 4
OUT_PAD = 32

_GDN = lax.GatherDimensionNumbers(
    offset_dims=(), collapsed_slice_dims=(0,), start_index_map=(0,))


def _perm(v, idx):
    return lax.gather(v, idx[:, None], _GDN, slice_sizes=(1,),
                      mode=lax.GatherScatterMode.PROMISE_IN_BOUNDS)


def _body(doc_idx_hbm, word_idx_hbm, doc_table_hbm, word_table_hbm, out_hbm,
          idx_w_all, idx_d_all,
          rows_d0, rows_d1, rows_d2, rows_d3,
          rows_w0, rows_w1, rows_w2, rows_w3,
          out_v0, out_v1, out_v2, out_v3,
          sem0, sem1, sem2, sem3,
          dsem0, dsem1, dsem2, dsem3):
    wid = lax.axis_index("s") * NC + lax.axis_index("c")
    wbase = wid * PER_W

    rows_d = (rows_d0, rows_d1, rows_d2, rows_d3)
    rows_w = (rows_w0, rows_w1, rows_w2, rows_w3)
    out_v = (out_v0, out_v1, out_v2, out_v3)
    sem = (sem0, sem1, sem2, sem3)
    dsem = (dsem0, dsem1, dsem2, dsem3)

    pltpu.sync_copy(
        word_idx_hbm.at[pl.ds(pl.multiple_of(wbase * W, 8), PER_W * W)],
        idx_w_all)
    pltpu.sync_copy(doc_idx_hbm.at[pl.ds(pl.multiple_of(wbase, 8), PER_W)],
                    idx_d_all.at[pl.ds(0, PER_W)])

    def issue(g, b):
        off = 0
        for n in SLABS:
            pltpu.make_async_copy(
                word_table_hbm.at[idx_w_all.at[pl.ds(g * IDX_PER_CHUNK + off,
                                                     n)]],
                rows_w[b].at[pl.ds(off, n)],
                sem[b]).start()
            off += n
        def dma_row(r, carry):
            v = idx_w_all[pl.ds(g * IDX_PER_CHUNK + STREAM_W + r * LANES,
                                LANES)]
            for j in range(LANES):
                pltpu.make_async_copy(
                    word_table_hbm.at[pl.ds(v[j], 1)],
                    rows_w[b].at[pl.ds(STREAM_W + r * LANES + j, 1)],
                    dsem[b]).start()
            return carry

        lax.fori_loop(0, DMA_W // LANES, dma_row, jnp.int32(0))

        vd = idx_d_all[pl.ds(g * C, LANES)]
        for j in range(C):
            pltpu.make_async_copy(
                doc_table_hbm.at[pl.ds(vd[j], 1)],
                rows_d[b].at[pl.ds(j, 1)],
                dsem[b]).start()

    def drain(g, b):
        off = 0
        for n in SLABS:
            pltpu.make_async_copy(
                word_table_hbm.at[idx_w_all.at[pl.ds(g * IDX_PER_CHUNK + off,
                                                     n)]],
                rows_w[b].at[pl.ds(off, n)],
                sem[b]).wait()
            off += n

        def dma_row_wait(r, carry):
            pltpu.make_async_copy(
                doc_table_hbm.at[pl.ds(jnp.int32(0), 1)],
                rows_d[b].at[pl.ds(0, 1)],
                dsem[b]).wait()
            return carry

        lax.fori_loop(0, DMA_W + C, dma_row_wait, jnp.int32(0))

    inv_n = jnp.float32(1.0 / (L + 1))
    lane = lax.broadcasted_iota(jnp.int32, (LANES,), 0)
    rot_idx = {d: (lane + d) & (LANES - 1) for d in (1, 2, 4, 8)}

    def compute(g, b):
        rd, rw, ov = rows_d[b], rows_w[b], out_v[b]

        def ex_body(i, carry):
            rb = i * W
            m = [rd[i, pl.ds(LANES * e, LANES)] for e in range(EV)]
            for j in range(L):
                for e in range(EV):
                    m[e] = m[e] + rw[rb + j, pl.ds(LANES * e, LANES)]
            m = [v * inv_n for v in m]
            acc_a = jnp.zeros((LANES,), jnp.float32)
            acc_b = jnp.zeros((LANES,), jnp.float32)
            for k in range(K):
                r = rb + L + k
                p = m[0] * rw[r, pl.ds(0, LANES)]
                for e in range(1, EV):
                    p = p + m[e] * rw[r, pl.ds(LANES * e, LANES)]
                for d in (8, 4, 2, 1):
                    p = p + _perm(p, rot_idx[d])
                if k < LANES:
                    acc_a = jnp.where(lane == k, p, acc_a)
                else:
                    acc_b = jnp.where(lane == (k - LANES), p, acc_b)
            ov[i, pl.ds(0, LANES)] = acc_a
            ov[i, pl.ds(LANES, LANES)] = acc_b
            return carry

        lax.fori_loop(0, C, ex_body, jnp.int32(0))
        base = pl.multiple_of(wbase + g * C, 8)
        pltpu.sync_copy(ov, out_hbm.at[pl.ds(base, C)])

    for b in range(NBUF):
        issue(b, b)

    def it_body(it, carry):
        for b in range(NBUF):
            g = NBUF * it + b
            drain(g, b)
            compute(g, b)

            @pl.when(g + NBUF < CHUNKS)
            def _():
                issue(g + NBUF, b)
        return carry

    lax.fori_loop(0, CHUNKS // NBUF, it_body, jnp.int32(0))


@jax.jit
def _pv_dm(doc_idx, word_idx, doc_table, word_table):
    mesh = plsc.VectorSubcoreMesh(core_axis_name="c", subcore_axis_name="s")
    fn = pl.kernel(
        _body,
        mesh=mesh,
        compiler_params=pltpu.CompilerParams(use_tc_tiling_on_sc=False),
        out_type=jax.ShapeDtypeStruct((B, OUT_PAD), jnp.float32),
        scratch_types=[
            pltpu.VMEM((PER_W * W,), jnp.int32),
            pltpu.VMEM((PER_W + LANES,), jnp.int32),
            pltpu.VMEM((C, E), jnp.float32),
            pltpu.VMEM((C, E), jnp.float32),
            pltpu.VMEM((C, E), jnp.float32),
            pltpu.VMEM((C, E), jnp.float32),
            pltpu.VMEM((IDX_PER_CHUNK, E), jnp.float32),
            pltpu.VMEM((IDX_PER_CHUNK, E), jnp.float32),
            pltpu.VMEM((IDX_PER_CHUNK, E), jnp.float32),
            pltpu.VMEM((IDX_PER_CHUNK, E), jnp.float32),
            pltpu.VMEM((C, OUT_PAD), jnp.float32),
            pltpu.VMEM((C, OUT_PAD), jnp.float32),
            pltpu.VMEM((C, OUT_PAD), jnp.float32),
            pltpu.VMEM((C, OUT_PAD), jnp.float32),
            pltpu.SemaphoreType.DMA,
            pltpu.SemaphoreType.DMA,
            pltpu.SemaphoreType.DMA,
            pltpu.SemaphoreType.DMA,
            pltpu.SemaphoreType.DMA,
            pltpu.SemaphoreType.DMA,
            pltpu.SemaphoreType.DMA,
            pltpu.SemaphoreType.DMA,
        ],
    )
    return fn(doc_idx, word_idx, doc_table, word_table)


def kernel(doc_ids, context_ids, target_noise_ids, doc_table, word_table):
    doc_idx = doc_ids.reshape(B).astype(jnp.int32)
    word_idx = jnp.concatenate(
        [context_ids, target_noise_ids], axis=1).astype(jnp.int32)
    word_idx = word_idx.reshape(B * W)
    out = _pv_dm(doc_idx, word_idx, doc_table, word_table)
    return out[:, :K]


# speedup vs baseline: 1.0802x; 1.0000x over previous
import functools

import jax
import jax.numpy as jnp
from jax import lax
from jax.experimental import pallas as pl
from jax.experimental.pallas import tpu as pltpu
from jax.experimental.pallas import tpu_sc as plsc

NC = 2
NS = 16
LANES = 16

B = 16384
L = 20
K = 20
E = 64
W = L + K
EV = E // LANES

NW = NC * NS
PER_W = B // NW
C = 8
CHUNKS = PER_W // C
IDX_PER_CHUNK = C * W
STREAM_W = 320
DMA_W = IDX_PER_CHUNK - STREAM_W
SLABS = (64, 64, 64, 64, 64)
NBUF = 4
OUT_PAD = 32

_GDN = lax.GatherDimensionNumbers(
    offset_dims=(), collapsed_slice_dims=(0,), start_index_map=(0,))


def _perm(v, idx):
    return lax.gather(v, idx[:, None], _GDN, slice_sizes=(1,),
                      mode=lax.GatherScatterMode.PROMISE_IN_BOUNDS)


def _body(doc_idx_hbm, word_idx_hbm, doc_table_hbm, word_table_hbm, out_hbm,
          idx_w_all, idx_d_all,
          rows_d0, rows_d1, rows_d2, rows_d3,
          rows_w0, rows_w1, rows_w2, rows_w3,
          out_v0, out_v1, out_v2, out_v3,
          sem0, sem1, sem2, sem3,
          dsem0, dsem1, dsem2, dsem3):
    wid = lax.axis_index("s") * NC + lax.axis_index("c")
    wbase = wid * PER_W

    rows_d = (rows_d0, rows_d1, rows_d2, rows_d3)
    rows_w = (rows_w0, rows_w1, rows_w2, rows_w3)
    out_v = (out_v0, out_v1, out_v2, out_v3)
    sem = (sem0, sem1, sem2, sem3)
    dsem = (dsem0, dsem1, dsem2, dsem3)

    pltpu.sync_copy(
        word_idx_hbm.at[pl.ds(pl.multiple_of(wbase * W, 8), PER_W * W)],
        idx_w_all)
    pltpu.sync_copy(doc_idx_hbm.at[pl.ds(pl.multiple_of(wbase, 8), PER_W)],
                    idx_d_all.at[pl.ds(0, PER_W)])

    def issue(g, b):
        off = 0
        for n in SLABS:
            pltpu.make_async_copy(
                word_table_hbm.at[idx_w_all.at[pl.ds(g * IDX_PER_CHUNK + off,
                                                     n)]],
                rows_w[b].at[pl.ds(off, n)],
                sem[b]).start()
            off += n
        def dma_row(r, carry):
            v = idx_w_all[pl.ds(g * IDX_PER_CHUNK + STREAM_W + r * LANES,
                                LANES)]
            for j in range(LANES):
                pltpu.make_async_copy(
                    word_table_hbm.at[pl.ds(v[j], 1)],
                    rows_w[b].at[pl.ds(STREAM_W + r * LANES + j, 1)],
                    dsem[b]).start()
            return carry

        lax.fori_loop(0, DMA_W // LANES, dma_row, jnp.int32(0))

        vd = idx_d_all[pl.ds(g * C, LANES)]
        for j in range(C):
            pltpu.make_async_copy(
                doc_table_hbm.at[pl.ds(vd[j], 1)],
                rows_d[b].at[pl.ds(j, 1)],
                dsem[b]).start()

    def drain(g, b):
        off = 0
        for n in SLABS:
            pltpu.make_async_copy(
                word_table_hbm.at[idx_w_all.at[pl.ds(g * IDX_PER_CHUNK + off,
                                                     n)]],
                rows_w[b].at[pl.ds(off, n)],
                sem[b]).wait()
            off += n

        def dma_row_wait(r, carry):
            pltpu.make_async_copy(
                doc_table_hbm.at[pl.ds(jnp.int32(0), 1)],
                rows_d[b].at[pl.ds(0, 1)],
                dsem[b]).wait()
            return carry

        lax.fori_loop(0, DMA_W + C, dma_row_wait, jnp.int32(0))

    inv_n = jnp.float32(1.0 / (L + 1))
    lane = lax.broadcasted_iota(jnp.int32, (LANES,), 0)
    rot_idx = {d: (lane + d) & (LANES - 1) for d in (1, 2, 4, 8)}

    def compute(g, b):
        rd, rw, ov = rows_d[b], rows_w[b], out_v[b]

        def ex_body(i, carry):
            rb = i * W
            m = [rd[i, pl.ds(LANES * e, LANES)] for e in range(EV)]
            for j in range(L):
                for e in range(EV):
                    m[e] = m[e] + rw[rb + j, pl.ds(LANES * e, LANES)]
            m = [v * inv_n for v in m]
            acc_a = jnp.zeros((LANES,), jnp.float32)
            acc_b = jnp.zeros((LANES,), jnp.float32)
            for k in range(K):
                r = rb + L + k
                p = m[0] * rw[r, pl.ds(0, LANES)]
                for e in range(1, EV):
                    p = p + m[e] * rw[r, pl.ds(LANES * e, LANES)]
                for d in (8, 4, 2, 1):
                    p = p + _perm(p, rot_idx[d])
                if k < LANES:
                    acc_a = jnp.where(lane == k, p, acc_a)
                else:
                    acc_b = jnp.where(lane == (k - LANES), p, acc_b)
            ov[i, pl.ds(0, LANES)] = acc_a
            ov[i, pl.ds(LANES, LANES)] = acc_b
            return carry

        lax.fori_loop(0, C, ex_body, jnp.int32(0))
        base = pl.multiple_of(wbase + g * C, 8)
        pltpu.sync_copy(ov, out_hbm.at[pl.ds(base, C)])

    for b in range(NBUF):
        issue(b, b)

    def it_body(it, carry):
        for b in range(NBUF):
            g = NBUF * it + b
            drain(g, b)
            compute(g, b)

            @pl.when(g + NBUF < CHUNKS)
            def _():
                issue(g + NBUF, b)
        return carry

    lax.fori_loop(0, CHUNKS // NBUF, it_body, jnp.int32(0))


@jax.jit
def _pv_dm(doc_idx, word_idx, doc_table, word_table):
    mesh = plsc.VectorSubcoreMesh(core_axis_name="c", subcore_axis_name="s")
    fn = pl.kernel(
        _body,
        mesh=mesh,
        compiler_params=pltpu.CompilerParams(use_tc_tiling_on_sc=False),
        out_type=jax.ShapeDtypeStruct((B, OUT_PAD), jnp.float32),
        scratch_types=[
            pltpu.VMEM((PER_W * W,), jnp.int32),
            pltpu.VMEM((PER_W + LANES,), jnp.int32),
            pltpu.VMEM((C, E), jnp.float32),
            pltpu.VMEM((C, E), jnp.float32),
            pltpu.VMEM((C, E), jnp.float32),
            pltpu.VMEM((C, E), jnp.float32),
            pltpu.VMEM((IDX_PER_CHUNK, E), jnp.float32),
            pltpu.VMEM((IDX_PER_CHUNK, E), jnp.float32),
            pltpu.VMEM((IDX_PER_CHUNK, E), jnp.float32),
            pltpu.VMEM((IDX_PER_CHUNK, E), jnp.float32),
            pltpu.VMEM((C, OUT_PAD), jnp.float32),
            pltpu.VMEM((C, OUT_PAD), jnp.float32),
            pltpu.VMEM((C, OUT_PAD), jnp.float32),
            pltpu.VMEM((C, OUT_PAD), jnp.float32),
            pltpu.SemaphoreType.DMA,
            pltpu.SemaphoreType.DMA,
            pltpu.SemaphoreType.DMA,
            pltpu.SemaphoreType.DMA,
            pltpu.SemaphoreType.DMA,
            pltpu.SemaphoreType.DMA,
            pltpu.SemaphoreType.DMA,
            pltpu.SemaphoreType.DMA,
        ],
    )
    return fn(doc_idx, word_idx, doc_table, word_table)


def kernel(doc_ids, context_ids, target_noise_ids, doc_table, word_table):
    doc_idx = doc_ids.reshape(B).astype(jnp.int32)
    word_idx = jnp.concatenate(
        [context_ids, target_noise_ids], axis=1).astype(jnp.int32)
    word_idx = word_idx.reshape(B * W)
    out = _pv_dm(doc_idx, word_idx, doc_table, word_table)
    return out[:, :K]
